# baseline (speedup 1.0000x reference)
import functools
import numpy as np
import jax
import jax.numpy as jnp
from jax import lax
from jax.experimental import pallas as pl
from jax.experimental.pallas import tpu as pltpu, tpu_sc as plsc

_B, _H, _W = 2, 192, 640
_R = 20
_K = 41 * 41
_KPAD = 1696
_RP = 2
_EDGE = 0.95
_PW = _W + 2 * _R
_PH = _H + 2 * _R
_NW = 32
_RPT = (_B * _H) // _NW
_DROWS = _RPT + 2 * _R

_SMOOTH_W = [
    [float(np.exp(-np.sqrt(dx * dx + dy * dy) * 0.7)) for dx in range(-_RP, _RP + 1)]
    for dy in range(-_RP, _RP + 1)
]

_INTERPRET = False


def _iota16():
    return lax.iota(jnp.int32, 16)


def _sc_search(pdst_hbm, srcm_hbm, doff_hbm, cxf_hbm, cyf_hbm,
               dxm_hbm, dym_hbm, fm_hbm,
               dstbuf, srcbuf, doffbuf, cxfbuf, cyfbuf, odx, ody, ofd, dmasem):
    wid = lax.axis_index("s") * 2 + lax.axis_index("c")
    b = wid // 16
    r0 = (wid % 16) * _RPT

    c1 = pltpu.async_copy(pdst_hbm.at[pl.ds(b * _PH * _PW + r0 * _PW, _DROWS * _PW)], dstbuf, dmasem)
    c2 = pltpu.async_copy(srcm_hbm.at[pl.ds(b * _H * _W + r0 * _W, _RPT * _W)], srcbuf, dmasem)
    c3 = pltpu.async_copy(doff_hbm, doffbuf, dmasem)
    c4 = pltpu.async_copy(cxf_hbm, cxfbuf, dmasem)
    c5 = pltpu.async_copy(cyf_hbm, cyfbuf, dmasem)
    c1.wait(); c2.wait(); c3.wait(); c4.wait(); c5.wait()

    iota = _iota16()
    zeros = jnp.zeros((16,), jnp.float32)

    def rowbody(yl, _):
        rowbase = (yl + _R) * _PW + _R

        def gbody(xi, _):
            x0 = xi * 16
            srcv = srcbuf[pl.ds(yl * _W + x0, 16)]
            m0 = srcv > _EDGE

            def lane_cond(c):
                return jnp.any(c[0])

            def lane_body(c, rowbase=rowbase, x0=x0):
                m, vdx, vdy, vf = c
                jv = plsc.all_reduce_ffs(m)
                pbase = rowbase + x0 + jv

                kmin = None
                for t in range(4):
                    dof = doffbuf[pl.ds(16 * t, 16)]
                    dv = plsc.load_gather(dstbuf, [pbase + dof])
                    kt = jnp.where(dv > _EDGE, 16 * t + iota, 99999)
                    kmin = kt if kmin is None else jnp.minimum(kmin, kt)
                hk64 = jnp.min(kmin)
                found64 = hk64 < 99999

                def pcond(c2):
                    return jnp.logical_not(c2[1]) & (c2[0] < _KPAD)

                def pbody(c2):
                    k0, done, hk = c2
                    dofa = doffbuf[pl.ds(k0, 16)]
                    dofb = doffbuf[pl.ds(k0 + 16, 16)]
                    kva = k0 + iota
                    kvb = kva + 16
                    dva = plsc.load_gather(dstbuf, [pbase + dofa],
                                           mask=kva < _K)
                    dvb = plsc.load_gather(dstbuf, [pbase + dofb],
                                           mask=kvb < _K)
                    ka = jnp.where((dva > _EDGE) & (kva < _K), kva, 99999)
                    kb = jnp.where((dvb > _EDGE) & (kvb < _K), kvb, 99999)
                    hk2 = jnp.min(jnp.minimum(ka, kb))
                    hit = hk2 < 99999
                    return (k0 + 32, hit, jnp.where(hit, hk2, hk))

                _, done, hkt = lax.while_loop(pcond, pbody, (64, found64, 0))
                done = found64 | done
                hk = jnp.where(found64, hk64, hkt)
                hkv = jnp.full((16,), jnp.where(done, hk, 0), jnp.int32)
                dxs = plsc.load_gather(cxfbuf, [hkv])
                dys = plsc.load_gather(cyfbuf, [hkv])
                lanesel = iota == jv
                hitsel = lanesel & done
                vf = jnp.where(hitsel, 1.0, vf)
                vdx = jnp.where(hitsel, dxs, vdx)
                vdy = jnp.where(hitsel, dys, vdy)
                return (m & jnp.logical_not(lanesel), vdx, vdy, vf)

            _, vdx, vdy, vf = lax.while_loop(
                lane_cond, lane_body, (m0, zeros, zeros, zeros))
            off = yl * _W + x0
            odx[pl.ds(off, 16)] = vdx
            ody[pl.ds(off, 16)] = vdy
            ofd[pl.ds(off, 16)] = vf
            return 0

        lax.fori_loop(0, _W // 16, gbody, 0)
        return 0

    lax.fori_loop(0, _RPT, rowbody, 0)

    o1 = pltpu.async_copy(odx, dxm_hbm.at[pl.ds(b * _H * _W + r0 * _W, _RPT * _W)], dmasem)
    o2 = pltpu.async_copy(ody, dym_hbm.at[pl.ds(b * _H * _W + r0 * _W, _RPT * _W)], dmasem)
    o3 = pltpu.async_copy(ofd, fm_hbm.at[pl.ds(b * _H * _W + r0 * _W, _RPT * _W)], dmasem)
    o1.wait(); o2.wait(); o3.wait()


def sc_search(pdst, srcm, doff, cxf, cyf):
    mesh = plsc.VectorSubcoreMesh(core_axis_name="c", subcore_axis_name="s",
                                  num_cores=2, num_subcores=16)
    out = jax.ShapeDtypeStruct((_B * _H * _W,), jnp.float32)
    f = pl.kernel(
        _sc_search,
        out_type=[out, out, out],
        mesh=mesh,
        scratch_types=[
            pltpu.VMEM((_DROWS * _PW,), jnp.float32),
            pltpu.VMEM((_RPT * _W,), jnp.float32),
            pltpu.VMEM((_KPAD,), jnp.int32),
            pltpu.VMEM((_KPAD,), jnp.float32),
            pltpu.VMEM((_KPAD,), jnp.float32),
            pltpu.VMEM((_RPT * _W,), jnp.float32),
            pltpu.VMEM((_RPT * _W,), jnp.float32),
            pltpu.VMEM((_RPT * _W,), jnp.float32),
            pltpu.SemaphoreType.DMA,
        ],
        compiler_params=pltpu.CompilerParams(needs_layout_passes=False),
        interpret=_INTERPRET,
    )
    return f(pdst, srcm, doff, cxf, cyf)


def _tc_smooth_kernel(dx_ref, dy_ref, f_ref, mx_ref, my_ref, ox_ref, oy_ref, cx_ref, cy_ref):
    dispx = dx_ref[0]
    dispy = dy_ref[0]
    foundf = f_ref[0]
    H, W = dispx.shape

    xg = lax.broadcasted_iota(jnp.int32, (H, W), 1).astype(jnp.float32)
    yg = lax.broadcasted_iota(jnp.int32, (H, W), 0).astype(jnp.float32)

    ox_ref[0] = xg * foundf
    oy_ref[0] = yg * foundf
    cx_ref[0] = (xg + dispx) * foundf
    cy_ref[0] = (yg + dispy) * foundf

    pdx = jnp.pad(dispx, _RP)
    pdy = jnp.pad(dispy, _RP)
    pm = jnp.pad(foundf, _RP)
    numx = jnp.zeros((H, W), jnp.float32)
    numy = jnp.zeros((H, W), jnp.float32)
    den = jnp.zeros((H, W), jnp.float32)
    for dy in range(-_RP, _RP + 1):
        for dx in range(-_RP, _RP + 1):
            w = _SMOOTH_W[dy + _RP][dx + _RP]
            numx = numx + w * pdx[_RP + dy:_RP + dy + H, _RP + dx:_RP + dx + W]
            numy = numy + w * pdy[_RP + dy:_RP + dy + H, _RP + dx:_RP + dx + W]
            den = den + w * pm[_RP + dy:_RP + dy + H, _RP + dx:_RP + dx + W]

    mx_ref[0] = xg + numx * 1.9 / (den * 24.0 / 24.0 + 1.6)
    my_ref[0] = yg + numy * 1.9 / (den + 1.6)


def tc_smooth(dispx, dispy, foundf):
    out = jax.ShapeDtypeStruct((_B, _H, _W), jnp.float32)
    spec = pl.BlockSpec((1, _H, _W), lambda b: (b, 0, 0))
    return pl.pallas_call(
        _tc_smooth_kernel,
        grid=(_B,),
        in_specs=[spec] * 3,
        out_specs=[spec] * 6,
        out_shape=[out] * 6,
        interpret=_INTERPRET,
    )(dispx, dispy, foundf)


def _offsets():
    span = np.arange(-_R, _R + 1)
    xx, yy = np.meshgrid(span, span)
    xx = xx.flatten().astype(np.float32)
    yy = yy.flatten().astype(np.float32)
    idx = np.argsort(xx ** 2 + yy ** 2, kind='stable')
    xx, yy = xx[idx], yy[idx]
    doff = (yy.astype(np.int64) * _PW + xx.astype(np.int64)).astype(np.int32)
    doff = np.concatenate([doff, np.zeros(_KPAD - _K, np.int32)])
    cxf = np.concatenate([xx, np.zeros(_KPAD - _K, np.float32)])
    cyf = np.concatenate([yy, np.zeros(_KPAD - _K, np.float32)])
    return jnp.asarray(doff), jnp.asarray(cxf), jnp.asarray(cyf)


def kernel(binMapsrc, binMapdst, xx, yy, sxx, syy, cxx, cyy):
    B, C, H, W = binMapsrc.shape
    doff, cxf, cyf = _offsets()
    pdst = jnp.pad(binMapdst.reshape(B, H, W), ((0, 0), (_R, _R), (_R, _R)))
    pdst = pdst.reshape(B * _PH * _PW)
    srcm = binMapsrc.reshape(B * H * W)
    dispx, dispy, foundf = sc_search(pdst, srcm, doff, cxf, cyf)
    outs = tc_smooth(dispx.reshape(B, H, W), dispy.reshape(B, H, W),
                     foundf.reshape(B, H, W))
    return tuple(o.reshape(B, C, H, W) for o in outs)


# speedup vs baseline: 1.0015x; 1.0015x over previous
import numpy as np
import jax
import jax.numpy as jnp
from jax import lax
from jax.experimental import pallas as pl
from jax.experimental.pallas import tpu as pltpu, tpu_sc as plsc

_B, _H, _W = 2, 192, 640
_R = 20
_K = 41 * 41
_KPAD = 1696
_RP = 2
_EDGE = 0.95
_PW = _W + 2 * _R
_PH = _H + 2 * _R
_NW = 32
_RPT = (_B * _H) // _NW
_DROWS = _RPT + 2 * _R

_SMOOTH_W = [
    [float(np.exp(-np.sqrt(dx * dx + dy * dy) * 0.7)) for dx in range(-_RP, _RP + 1)]
    for dy in range(-_RP, _RP + 1)
]


def _iota16():
    return lax.iota(jnp.int32, 16)


def _sc_search(pdst_hbm, srcm_hbm, doff_hbm, cxf_hbm, cyf_hbm,
               dxm_hbm, dym_hbm, fm_hbm,
               dstbuf, srcbuf, doffbuf, cxfbuf, cyfbuf, odx, ody, ofd, dmasem):
    wid = lax.axis_index("s") * 2 + lax.axis_index("c")
    b = wid // 16
    r0 = (wid % 16) * _RPT

    c1 = pltpu.async_copy(pdst_hbm.at[pl.ds(b * _PH * _PW + r0 * _PW, _DROWS * _PW)], dstbuf, dmasem)
    c2 = pltpu.async_copy(srcm_hbm.at[pl.ds(b * _H * _W + r0 * _W, _RPT * _W)], srcbuf, dmasem)
    c3 = pltpu.async_copy(doff_hbm, doffbuf, dmasem)
    c4 = pltpu.async_copy(cxf_hbm, cxfbuf, dmasem)
    c5 = pltpu.async_copy(cyf_hbm, cyfbuf, dmasem)
    c1.wait(); c2.wait(); c3.wait(); c4.wait(); c5.wait()

    iota = _iota16()
    zeros = jnp.zeros((16,), jnp.float32)

    def rowbody(yl, _):
        rowbase = (yl + _R) * _PW + _R

        def gbody(xi, _):
            x0 = xi * 16
            srcv = srcbuf[pl.ds(yl * _W + x0, 16)]
            m0 = srcv > _EDGE

            def lane_cond(c):
                return jnp.any(c[0])

            def lane_body(c, rowbase=rowbase, x0=x0):
                m, vdx, vdy, vf = c
                jv = plsc.all_reduce_ffs(m)
                pbase = rowbase + x0 + jv

                kmin = None
                for t in range(4):
                    dof = doffbuf[pl.ds(16 * t, 16)]
                    dv = plsc.load_gather(dstbuf, [pbase + dof])
                    kt = jnp.where(dv > _EDGE, 16 * t + iota, 99999)
                    kmin = kt if kmin is None else jnp.minimum(kmin, kt)
                hk64 = jnp.min(kmin)
                found64 = hk64 < 99999

                def pcond(c2):
                    return jnp.logical_not(c2[1]) & (c2[0] < _KPAD)

                def pbody(c2):
                    k0, done, hk = c2
                    dofa = doffbuf[pl.ds(k0, 16)]
                    dofb = doffbuf[pl.ds(k0 + 16, 16)]
                    kva = k0 + iota
                    kvb = kva + 16
                    dva = plsc.load_gather(dstbuf, [pbase + dofa],
                                           mask=kva < _K)
                    dvb = plsc.load_gather(dstbuf, [pbase + dofb],
                                           mask=kvb < _K)
                    ka = jnp.where((dva > _EDGE) & (kva < _K), kva, 99999)
                    kb = jnp.where((dvb > _EDGE) & (kvb < _K), kvb, 99999)
                    hk2 = jnp.min(jnp.minimum(ka, kb))
                    hit = hk2 < 99999
                    return (k0 + 32, hit, jnp.where(hit, hk2, hk))

                _, done, hkt = lax.while_loop(pcond, pbody, (64, found64, 0))
                done = found64 | done
                hk = jnp.where(found64, hk64, hkt)
                hkv = jnp.full((16,), jnp.where(done, hk, 0), jnp.int32)
                dxs = plsc.load_gather(cxfbuf, [hkv])
                dys = plsc.load_gather(cyfbuf, [hkv])
                lanesel = iota == jv
                hitsel = lanesel & done
                vf = jnp.where(hitsel, 1.0, vf)
                vdx = jnp.where(hitsel, dxs, vdx)
                vdy = jnp.where(hitsel, dys, vdy)
                return (m & jnp.logical_not(lanesel), vdx, vdy, vf)

            _, vdx, vdy, vf = lax.while_loop(
                lane_cond, lane_body, (m0, zeros, zeros, zeros))
            off = yl * _W + x0
            odx[pl.ds(off, 16)] = vdx
            ody[pl.ds(off, 16)] = vdy
            ofd[pl.ds(off, 16)] = vf
            return 0

        lax.fori_loop(0, _W // 16, gbody, 0)
        return 0

    lax.fori_loop(0, _RPT, rowbody, 0)

    o1 = pltpu.async_copy(odx, dxm_hbm.at[pl.ds(b * _H * _W + r0 * _W, _RPT * _W)], dmasem)
    o2 = pltpu.async_copy(ody, dym_hbm.at[pl.ds(b * _H * _W + r0 * _W, _RPT * _W)], dmasem)
    o3 = pltpu.async_copy(ofd, fm_hbm.at[pl.ds(b * _H * _W + r0 * _W, _RPT * _W)], dmasem)
    o1.wait(); o2.wait(); o3.wait()


def sc_search(pdst, srcm, doff, cxf, cyf):
    mesh = plsc.VectorSubcoreMesh(core_axis_name="c", subcore_axis_name="s",
                                  num_cores=2, num_subcores=16)
    out = jax.ShapeDtypeStruct((_B * _H * _W,), jnp.float32)
    f = pl.kernel(
        _sc_search,
        out_type=[out, out, out],
        mesh=mesh,
        scratch_types=[
            pltpu.VMEM((_DROWS * _PW,), jnp.float32),
            pltpu.VMEM((_RPT * _W,), jnp.float32),
            pltpu.VMEM((_KPAD,), jnp.int32),
            pltpu.VMEM((_KPAD,), jnp.float32),
            pltpu.VMEM((_KPAD,), jnp.float32),
            pltpu.VMEM((_RPT * _W,), jnp.float32),
            pltpu.VMEM((_RPT * _W,), jnp.float32),
            pltpu.VMEM((_RPT * _W,), jnp.float32),
            pltpu.SemaphoreType.DMA,
        ],
        compiler_params=pltpu.CompilerParams(needs_layout_passes=False),
    )
    return f(pdst, srcm, doff, cxf, cyf)


def _tc_smooth_kernel(dx_ref, dy_ref, f_ref, mx_ref, my_ref, ox_ref, oy_ref, cx_ref, cy_ref):
    dispx = dx_ref[0]
    dispy = dy_ref[0]
    foundf = f_ref[0]
    H, W = dispx.shape

    xg = lax.broadcasted_iota(jnp.int32, (H, W), 1).astype(jnp.float32)
    yg = lax.broadcasted_iota(jnp.int32, (H, W), 0).astype(jnp.float32)

    ox_ref[0] = xg * foundf
    oy_ref[0] = yg * foundf
    cx_ref[0] = (xg + dispx) * foundf
    cy_ref[0] = (yg + dispy) * foundf

    pdx = jnp.pad(dispx, _RP)
    pdy = jnp.pad(dispy, _RP)
    pm = jnp.pad(foundf, _RP)
    numx = jnp.zeros((H, W), jnp.float32)
    numy = jnp.zeros((H, W), jnp.float32)
    den = jnp.zeros((H, W), jnp.float32)
    for dy in range(-_RP, _RP + 1):
        for dx in range(-_RP, _RP + 1):
            w = _SMOOTH_W[dy + _RP][dx + _RP]
            numx = numx + w * pdx[_RP + dy:_RP + dy + H, _RP + dx:_RP + dx + W]
            numy = numy + w * pdy[_RP + dy:_RP + dy + H, _RP + dx:_RP + dx + W]
            den = den + w * pm[_RP + dy:_RP + dy + H, _RP + dx:_RP + dx + W]

    mx_ref[0] = xg + numx * 1.9 / (den * 24.0 / 24.0 + 1.6)
    my_ref[0] = yg + numy * 1.9 / (den + 1.6)


def tc_smooth(dispx, dispy, foundf):
    out = jax.ShapeDtypeStruct((_B, _H, _W), jnp.float32)
    spec = pl.BlockSpec((1, _H, _W), lambda b: (b, 0, 0))
    return pl.pallas_call(
        _tc_smooth_kernel,
        grid=(_B,),
        in_specs=[spec] * 3,
        out_specs=[spec] * 6,
        out_shape=[out] * 6,
    )(dispx, dispy, foundf)


def _offsets():
    span = np.arange(-_R, _R + 1)
    xx, yy = np.meshgrid(span, span)
    xx = xx.flatten().astype(np.float32)
    yy = yy.flatten().astype(np.float32)
    idx = np.argsort(xx ** 2 + yy ** 2, kind='stable')
    xx, yy = xx[idx], yy[idx]
    doff = (yy.astype(np.int64) * _PW + xx.astype(np.int64)).astype(np.int32)
    doff = np.concatenate([doff, np.zeros(_KPAD - _K, np.int32)])
    cxf = np.concatenate([xx, np.zeros(_KPAD - _K, np.float32)])
    cyf = np.concatenate([yy, np.zeros(_KPAD - _K, np.float32)])
    return jnp.asarray(doff), jnp.asarray(cxf), jnp.asarray(cyf)


def kernel(binMapsrc, binMapdst, xx, yy, sxx, syy, cxx, cyy):
    B, C, H, W = binMapsrc.shape
    doff, cxf, cyf = _offsets()
    pdst = jnp.pad(binMapdst.reshape(B, H, W), ((0, 0), (_R, _R), (_R, _R)))
    pdst = pdst.reshape(B * _PH * _PW)
    srcm = binMapsrc.reshape(B * H * W)
    dispx, dispy, foundf = sc_search(pdst, srcm, doff, cxf, cyf)
    outs = tc_smooth(dispx.reshape(B, H, W), dispy.reshape(B, H, W),
                     foundf.reshape(B, H, W))
    return tuple(o.reshape(B, C, H, W) for o in outs)
